# baseline (speedup 1.0000x reference)
import functools

import jax
import jax.numpy as jnp
from jax import lax
from jax.experimental import pallas as pl
from jax.experimental.pallas import tpu as pltpu
from jax.experimental.pallas import tpu_sc as plsc

_N = 50000
_E = 300000
_D = 128
_OUT = 64
_EPS = 1e-5

_NC, _NS = 2, 16
_CH = 128
_CHUNKS = -(-_E // (_NS * _CH)) * _NS
_EPAD = _CHUNKS * _CH
_CPT = _CHUNKS // _NS
_K = 7
_NG = _CPT // _K
_NPAD = 50048
_RPT = _NPAD // _NS

_BR = 512
_GRID = -(-_N // _BR)

_mesh = plsc.VectorSubcoreMesh(
    core_axis_name="c", subcore_axis_name="s", num_cores=_NC, num_subcores=_NS
)
_sc_params = pltpu.CompilerParams(use_tc_tiling_on_sc=False)


@functools.partial(
    pl.kernel,
    out_type=jax.ShapeDtypeStruct((2, _NPAD, _D), jnp.float32),
    mesh=_mesh,
    scratch_types=[
        pltpu.VMEM_SHARED((_NPAD, 16), jnp.float32),
        pltpu.VMEM((_CH, 16), jnp.float32),
        pltpu.VMEM((_CPT, _CH), jnp.int32),
        pltpu.SemaphoreType.DMA,
    ],
    compiler_params=_sc_params,
)
def _sc_count(dcat, ones_h, z16, cnt_out, acc, ones_v, didx, sem):
    c = lax.axis_index("c")
    s = lax.axis_index("s")
    pltpu.sync_copy(ones_h, ones_v)
    pltpu.sync_copy(dcat.at[c, pl.ds(s * _CPT, _CPT)], didx)
    pltpu.sync_copy(z16, acc.at[pl.ds(s * _RPT, _RPT)])
    plsc.subcore_barrier()

    def body(t, carry):
        for b in range(_K):
            pltpu.async_copy(ones_v, acc.at[didx.at[t * _K + b]], sem,
                             add=True)
        for b in range(_K):
            pltpu.make_async_copy(z16.at[pl.ds(0, _CH)], ones_v, sem).wait()
        return carry

    lax.fori_loop(0, _NG, body, 0)
    plsc.subcore_barrier()
    pltpu.sync_copy(acc.at[pl.ds(s * _RPT, _RPT)],
                    cnt_out.at[c, pl.ds(s * _RPT, _RPT), pl.ds(0, 16)])


@functools.partial(
    pl.kernel,
    out_type=(jax.ShapeDtypeStruct((_NPAD, _D), jnp.bfloat16),
              jax.ShapeDtypeStruct((_NPAD, _D), jnp.bfloat16)),
    mesh=_mesh,
    scratch_types=[
        pltpu.VMEM_SHARED((_NPAD, 32), jnp.bfloat16),
        pltpu.VMEM((_RPT // 4, 32), jnp.bfloat16),
        pltpu.VMEM((_CPT, _CH), jnp.int32),
        pltpu.VMEM((_CPT, _CH), jnp.int32),
        [pltpu.VMEM((_CH, 32), jnp.bfloat16)] * (2 * _K),
        [pltpu.SemaphoreType.DMA] * 4,
    ],
    compiler_params=_sc_params,
)
def _sc_seg(hu4, hi4, s4_iu, d_iu, s4_ui, d_ui, zb, ssum_u, ssum_i,
            acc, zbuf, sidx, didx, rows, sems):
    c = lax.axis_index("c")
    s = lax.axis_index("s")
    rows_ab = (rows[:_K], rows[_K:])
    gsem = (sems[0], sems[1])
    ssem = (sems[2], sems[3])
    pltpu.sync_copy(zb.at[pl.ds(0, _RPT // 4)], zbuf)

    def fire_g(table, g, ab):
        for b in range(_K):
            pltpu.async_copy(table.at[sidx.at[g * _K + b]], rows_ab[ab][b],
                             gsem[ab])

    def drain_g(ab):
        for b in range(_K):
            pltpu.make_async_copy(zb.at[pl.ds(0, _CH)], rows_ab[ab][b],
                                  gsem[ab]).wait()

    def fire_s(g, ab):
        for b in range(_K):
            pltpu.async_copy(rows_ab[ab][b], acc.at[didx.at[g * _K + b]],
                             ssem[ab], add=True)

    def drain_s(ab):
        for b in range(_K):
            pltpu.make_async_copy(zb.at[pl.ds(0, _CH)], rows_ab[ab][b],
                                  ssem[ab]).wait()

    for rel in range(2):
        table = (hi4, hu4)[rel]
        s4 = (s4_iu, s4_ui)[rel]
        dd = (d_iu, d_ui)[rel]
        out = (ssum_u, ssum_i)[rel]
        pltpu.sync_copy(dd.at[pl.ds(s * _CPT, _CPT)], didx)
        if rel == 0:
            for z in range(4):
                pltpu.sync_copy(
                    zbuf, acc.at[pl.ds(s * _RPT + z * (_RPT // 4), _RPT // 4)])
        for p_loc in range(2):
            p = c * 2 + p_loc
            pltpu.sync_copy(s4.at[p, pl.ds(s * _CPT, _CPT)], sidx)
            plsc.subcore_barrier()

            fire_g(table, 0, 0)
            drain_g(0)
            fire_s(0, 0)
            fire_g(table, 1, 1)

            def pair(t, carry):
                for par in range(2):
                    g = 2 * t + 1 + par
                    ab = (1, 0)[par]
                    drain_g(ab)
                    fire_s(g, ab)
                    drain_s(1 - ab)
                    fire_g(table, g + 1, 1 - ab)
                return carry

            lax.fori_loop(0, (_NG - 3) // 2, pair, 0)
            drain_g(1)
            fire_s(_NG - 2, 1)
            drain_s(0)
            fire_g(table, _NG - 1, 0)
            drain_g(0)
            fire_s(_NG - 1, 0)
            drain_s(1)
            drain_s(0)
            plsc.subcore_barrier()
            pltpu.sync_copy(acc.at[pl.ds(s * _RPT, _RPT)],
                            out.at[pl.ds(s * _RPT, _RPT), pl.ds(p * 32, 32)])
            for z in range(4):
                pltpu.sync_copy(
                    zbuf, acc.at[pl.ds(s * _RPT + z * (_RPT // 4), _RPT // 4)])


def _proj_body(x_ref, w_ref, b_ref, o_ref, ob_ref):
    y = lax.dot_general(x_ref[...], w_ref[...], (((1,), (1,)), ((), ())),
                        preferred_element_type=jnp.float32)
    y = jnp.maximum(y + b_ref[...], 0.0)
    o_ref[...] = y
    ob_ref[...] = y.astype(jnp.bfloat16)


def _proj(x, W, b):
    return pl.pallas_call(
        _proj_body,
        grid=(_GRID,),
        in_specs=[pl.BlockSpec((_BR, _D), lambda i: (i, 0)),
                  pl.BlockSpec((_D, _D), lambda i: (0, 0)),
                  pl.BlockSpec((1, _D), lambda i: (0, 0))],
        out_specs=(pl.BlockSpec((_BR, _D), lambda i: (i, 0)),
                   pl.BlockSpec((_BR, _D), lambda i: (i, 0))),
        out_shape=(jax.ShapeDtypeStruct((_N, _D), jnp.float32),
                   jax.ShapeDtypeStruct((_N, _D), jnp.bfloat16)),
    )(x, W, b.reshape(1, _D))


def _layer_body(h_ref, ss_ref, cnt_ref, wl_ref, bl_ref, wr_ref, sc_ref,
                sh_ref, o_ref, ob_ref=None):
    h = h_ref[...]
    mean = (ss_ref[...].astype(jnp.float32)
            / jnp.maximum(cnt_ref[...][:, :1], 1.0))
    agg = lax.dot_general(mean, wl_ref[...], (((1,), (1,)), ((), ())),
                          preferred_element_type=jnp.float32)
    agg += lax.dot_general(h, wr_ref[...], (((1,), (1,)), ((), ())),
                           preferred_element_type=jnp.float32)
    t = (h + agg + bl_ref[...]) * 0.5
    y = jnp.maximum(t * sc_ref[...] + sh_ref[...], 0.0)
    o_ref[...] = y
    if ob_ref is not None:
        ob_ref[...] = y.astype(jnp.bfloat16)


def _layer(h, ss4, cnt, Wl, bl, Wr, scale, shift, want_bf16=True):
    if want_bf16:
        out_specs = (pl.BlockSpec((_BR, _D), lambda i: (i, 0)),
                     pl.BlockSpec((_BR, _D), lambda i: (i, 0)))
        out_shape = (jax.ShapeDtypeStruct((_N, _D), jnp.float32),
                     jax.ShapeDtypeStruct((_N, _D), jnp.bfloat16))
    else:
        out_specs = pl.BlockSpec((_BR, _D), lambda i: (i, 0))
        out_shape = jax.ShapeDtypeStruct((_N, _D), jnp.float32)
    return pl.pallas_call(
        _layer_body,
        grid=(_GRID,),
        in_specs=[pl.BlockSpec((_BR, _D), lambda i: (i, 0)),
                  pl.BlockSpec((_BR, _D), lambda i: (i, 0)),
                  pl.BlockSpec((_BR, _D), lambda i: (i, 0)),
                  pl.BlockSpec((_D, _D), lambda i: (0, 0)),
                  pl.BlockSpec((1, _D), lambda i: (0, 0)),
                  pl.BlockSpec((_D, _D), lambda i: (0, 0)),
                  pl.BlockSpec((1, _D), lambda i: (0, 0)),
                  pl.BlockSpec((1, _D), lambda i: (0, 0))],
        out_specs=out_specs,
        out_shape=out_shape,
    )(h, ss4, cnt, Wl, bl.reshape(1, _D), Wr, scale.reshape(1, _D),
      shift.reshape(1, _D))


def _head_body(h_ref, w_ref, b_ref, o_ref):
    y = lax.dot_general(h_ref[...], w_ref[...], (((1,), (1,)), ((), ())),
                        preferred_element_type=jnp.float32)
    o_ref[...] = y + b_ref[...]


def _head(h, Wh, bh):
    return pl.pallas_call(
        _head_body,
        grid=(_GRID,),
        in_specs=[pl.BlockSpec((_BR, _D), lambda i: (i, 0)),
                  pl.BlockSpec((_OUT, _D), lambda i: (0, 0)),
                  pl.BlockSpec((1, _OUT), lambda i: (0, 0))],
        out_specs=pl.BlockSpec((_BR, _OUT), lambda i: (i, 0)),
        out_shape=jax.ShapeDtypeStruct((_N, _OUT), jnp.float32),
    )(h, Wh, bh.reshape(1, _OUT))


def _prep_edges(src, dst):
    pad = _EPAD - _E
    srcp = jnp.concatenate([src, jnp.zeros((pad,), jnp.int32)])
    dstp = jnp.concatenate([dst, jnp.full((pad,), _N, jnp.int32)])
    s4 = (srcp[None, :] * 4
          + jnp.arange(4, dtype=jnp.int32)[:, None]).reshape(4, _CHUNKS, _CH)
    return s4, dstp.reshape(_CHUNKS, _CH)


def kernel(x_user, x_item, edge_index_ui, edge_index_iu, Wp_user, bp_user,
           Wp_item, bp_item, Wl0, bl0, Wr0, gamma0, beta0, Wl1, bl1, Wr1,
           gamma1, beta1, Wh, bh):
    ei_ui = edge_index_ui.astype(jnp.int32)
    ei_iu = edge_index_iu.astype(jnp.int32)
    s4_iu, d_iu = _prep_edges(ei_iu[0], ei_iu[1])
    s4_ui, d_ui = _prep_edges(ei_ui[0], ei_ui[1])
    dcat = jnp.stack([d_iu, d_ui])

    ones16 = jnp.ones((_CH, 16), jnp.float32)
    z16 = jnp.zeros((_RPT, 16), jnp.float32)
    zb = jnp.zeros((_RPT, 32), jnp.bfloat16)

    cnt = _sc_count(dcat, ones16, z16)
    cnt_u, cnt_i = cnt[0], cnt[1]

    hu, hub = _proj(x_user, Wp_user, bp_user)
    hi, hib = _proj(x_item, Wp_item, bp_item)

    inv = 1.0 / jnp.sqrt(1.0 + _EPS)
    one = jnp.ones((_D,), jnp.float32)
    zero = jnp.zeros((_D,), jnp.float32)

    for li, (Wl, bl, Wr, gm, bt) in enumerate(
            ((Wl0, bl0, Wr0, gamma0, beta0), (Wl1, bl1, Wr1, gamma1, beta1))):
        ssum_u, ssum_i = _sc_seg(hub.reshape(_N * 4, 32),
                                 hib.reshape(_N * 4, 32),
                                 s4_iu, d_iu, s4_ui, d_ui, zb)
        if li == 0:
            hu, hub = _layer(hu, ssum_u, cnt_u, Wl, bl, Wr, gm * inv, bt)
            hi, hib = _layer(hi, ssum_i, cnt_i, Wl, bl, Wr, one, zero)
        else:
            hu = _layer(hu, ssum_u, cnt_u, Wl, bl, Wr, gm * inv, bt,
                        want_bf16=False)
            hi = _layer(hi, ssum_i, cnt_i, Wl, bl, Wr, one, zero,
                        want_bf16=False)

    return _head(hu, Wh, bh), hu


# speedup vs baseline: 1.3466x; 1.3466x over previous
import functools

import jax
import jax.numpy as jnp
from jax import lax
from jax.experimental import pallas as pl
from jax.experimental.pallas import tpu as pltpu
from jax.experimental.pallas import tpu_sc as plsc

_N = 50000
_E = 300000
_D = 128
_OUT = 64
_EPS = 1e-5

_NC, _NS = 2, 16
_CH = 128
_CHUNKS = -(-_E // (_NS * _CH)) * _NS
_EPAD = _CHUNKS * _CH
_CPT = _CHUNKS // _NS
_K = 7
_NG = _CPT // _K
_NPAD = 50048
_RPT = _NPAD // _NS

_BR = 512
_GRID = -(-_N // _BR)

_mesh = plsc.VectorSubcoreMesh(
    core_axis_name="c", subcore_axis_name="s", num_cores=_NC, num_subcores=_NS
)
_sc_params = pltpu.CompilerParams(use_tc_tiling_on_sc=False)


@functools.partial(
    pl.kernel,
    out_type=jax.ShapeDtypeStruct((2, _NPAD, _D), jnp.float32),
    mesh=_mesh,
    scratch_types=[
        pltpu.VMEM_SHARED((_NPAD, 16), jnp.float32),
        pltpu.VMEM((_CH, 16), jnp.float32),
        pltpu.VMEM((_CPT, _CH), jnp.int32),
        pltpu.SemaphoreType.DMA,
    ],
    compiler_params=_sc_params,
)
def _sc_count(dcat, ones_h, z16, cnt_out, acc, ones_v, didx, sem):
    c = lax.axis_index("c")
    s = lax.axis_index("s")
    pltpu.sync_copy(ones_h, ones_v)
    pltpu.sync_copy(dcat.at[c, pl.ds(s * _CPT, _CPT)], didx)
    pltpu.sync_copy(z16, acc.at[pl.ds(s * _RPT, _RPT)])
    plsc.subcore_barrier()

    def body(t, carry):
        for b in range(_K):
            pltpu.async_copy(ones_v, acc.at[didx.at[t * _K + b]], sem,
                             add=True)
        for b in range(_K):
            pltpu.make_async_copy(z16.at[pl.ds(0, _CH)], ones_v, sem).wait()
        return carry

    lax.fori_loop(0, _NG, body, 0)
    plsc.subcore_barrier()
    pltpu.sync_copy(acc.at[pl.ds(s * _RPT, _RPT)],
                    cnt_out.at[c, pl.ds(s * _RPT, _RPT), pl.ds(0, 16)])


@functools.partial(
    pl.kernel,
    out_type=jax.ShapeDtypeStruct((_NPAD, _D), jnp.bfloat16),
    mesh=_mesh,
    scratch_types=[
        pltpu.VMEM_SHARED((_NPAD, 32), jnp.bfloat16),
        pltpu.VMEM((_RPT // 4, 32), jnp.bfloat16),
        pltpu.VMEM((_CPT, _CH), jnp.int32),
        pltpu.VMEM((_CPT, _CH), jnp.int32),
        [pltpu.VMEM((_CH, 32), jnp.bfloat16)] * (2 * _K),
        [pltpu.SemaphoreType.DMA] * 4,
    ],
    compiler_params=_sc_params,
)
def _sc_seg(table, s4, dd, zb, out,
            acc, zbuf, sidx, didx, rows, sems):
    c = lax.axis_index("c")
    s = lax.axis_index("s")
    rows_ab = (rows[:_K], rows[_K:])
    gsem = (sems[0], sems[1])
    ssem = (sems[2], sems[3])
    pltpu.sync_copy(zb.at[pl.ds(0, _RPT // 4)], zbuf)

    def fire_g(table, g, ab):
        for b in range(_K):
            pltpu.async_copy(table.at[sidx.at[g * _K + b]], rows_ab[ab][b],
                             gsem[ab])

    def drain_g(ab):
        for b in range(_K):
            pltpu.make_async_copy(zb.at[pl.ds(0, _CH)], rows_ab[ab][b],
                                  gsem[ab]).wait()

    def fire_s(g, ab):
        for b in range(_K):
            pltpu.async_copy(rows_ab[ab][b], acc.at[didx.at[g * _K + b]],
                             ssem[ab], add=True)

    def drain_s(ab):
        for b in range(_K):
            pltpu.make_async_copy(zb.at[pl.ds(0, _CH)], rows_ab[ab][b],
                                  ssem[ab]).wait()

    if True:
        pltpu.sync_copy(dd.at[pl.ds(s * _CPT, _CPT)], didx)
        for z in range(4):
            pltpu.sync_copy(
                zbuf, acc.at[pl.ds(s * _RPT + z * (_RPT // 4), _RPT // 4)])
        for p_loc in range(2):
            p = c * 2 + p_loc
            pltpu.sync_copy(s4.at[p, pl.ds(s * _CPT, _CPT)], sidx)
            plsc.subcore_barrier()

            fire_g(table, 0, 0)
            drain_g(0)
            fire_s(0, 0)
            fire_g(table, 1, 1)

            def pair(t, carry):
                for par in range(2):
                    g = 2 * t + 1 + par
                    ab = (1, 0)[par]
                    drain_g(ab)
                    fire_s(g, ab)
                    drain_s(1 - ab)
                    fire_g(table, g + 1, 1 - ab)
                return carry

            lax.fori_loop(0, (_NG - 3) // 2, pair, 0)
            drain_g(1)
            fire_s(_NG - 2, 1)
            drain_s(0)
            fire_g(table, _NG - 1, 0)
            drain_g(0)
            fire_s(_NG - 1, 0)
            drain_s(1)
            drain_s(0)
            plsc.subcore_barrier()
            pltpu.sync_copy(acc.at[pl.ds(s * _RPT, _RPT)],
                            out.at[pl.ds(s * _RPT, _RPT), pl.ds(p * 32, 32)])
            for z in range(4):
                pltpu.sync_copy(
                    zbuf, acc.at[pl.ds(s * _RPT + z * (_RPT // 4), _RPT // 4)])


def _proj_body(x_ref, w_ref, b_ref, o_ref, ob_ref):
    y = lax.dot_general(x_ref[...], w_ref[...], (((1,), (1,)), ((), ())),
                        preferred_element_type=jnp.float32)
    y = jnp.maximum(y + b_ref[...], 0.0)
    o_ref[...] = y
    ob_ref[...] = y.astype(jnp.bfloat16)


def _proj(x, W, b):
    return pl.pallas_call(
        _proj_body,
        grid=(_GRID,),
        in_specs=[pl.BlockSpec((_BR, _D), lambda i: (i, 0)),
                  pl.BlockSpec((_D, _D), lambda i: (0, 0)),
                  pl.BlockSpec((1, _D), lambda i: (0, 0))],
        out_specs=(pl.BlockSpec((_BR, _D), lambda i: (i, 0)),
                   pl.BlockSpec((_BR, _D), lambda i: (i, 0))),
        out_shape=(jax.ShapeDtypeStruct((_N, _D), jnp.float32),
                   jax.ShapeDtypeStruct((_N, _D), jnp.bfloat16)),
    )(x, W, b.reshape(1, _D))


def _layer_body(h_ref, ss_ref, cnt_ref, wl_ref, bl_ref, wr_ref, sc_ref,
                sh_ref, o_ref, ob_ref=None):
    h = h_ref[...]
    mean = (ss_ref[...].astype(jnp.float32)
            / jnp.maximum(cnt_ref[...][:, :1], 1.0))
    agg = lax.dot_general(mean, wl_ref[...], (((1,), (1,)), ((), ())),
                          preferred_element_type=jnp.float32)
    agg += lax.dot_general(h, wr_ref[...], (((1,), (1,)), ((), ())),
                           preferred_element_type=jnp.float32)
    t = (h + agg + bl_ref[...]) * 0.5
    y = jnp.maximum(t * sc_ref[...] + sh_ref[...], 0.0)
    o_ref[...] = y
    if ob_ref is not None:
        ob_ref[...] = y.astype(jnp.bfloat16)


def _layer(h, ss4, cnt, Wl, bl, Wr, scale, shift, want_bf16=True):
    if want_bf16:
        out_specs = (pl.BlockSpec((_BR, _D), lambda i: (i, 0)),
                     pl.BlockSpec((_BR, _D), lambda i: (i, 0)))
        out_shape = (jax.ShapeDtypeStruct((_N, _D), jnp.float32),
                     jax.ShapeDtypeStruct((_N, _D), jnp.bfloat16))
    else:
        out_specs = pl.BlockSpec((_BR, _D), lambda i: (i, 0))
        out_shape = jax.ShapeDtypeStruct((_N, _D), jnp.float32)
    return pl.pallas_call(
        _layer_body,
        grid=(_GRID,),
        in_specs=[pl.BlockSpec((_BR, _D), lambda i: (i, 0)),
                  pl.BlockSpec((_BR, _D), lambda i: (i, 0)),
                  pl.BlockSpec((_BR, _D), lambda i: (i, 0)),
                  pl.BlockSpec((_D, _D), lambda i: (0, 0)),
                  pl.BlockSpec((1, _D), lambda i: (0, 0)),
                  pl.BlockSpec((_D, _D), lambda i: (0, 0)),
                  pl.BlockSpec((1, _D), lambda i: (0, 0)),
                  pl.BlockSpec((1, _D), lambda i: (0, 0))],
        out_specs=out_specs,
        out_shape=out_shape,
    )(h, ss4, cnt, Wl, bl.reshape(1, _D), Wr, scale.reshape(1, _D),
      shift.reshape(1, _D))


def _head_body(h_ref, w_ref, b_ref, o_ref):
    y = lax.dot_general(h_ref[...], w_ref[...], (((1,), (1,)), ((), ())),
                        preferred_element_type=jnp.float32)
    o_ref[...] = y + b_ref[...]


def _head(h, Wh, bh):
    return pl.pallas_call(
        _head_body,
        grid=(_GRID,),
        in_specs=[pl.BlockSpec((_BR, _D), lambda i: (i, 0)),
                  pl.BlockSpec((_OUT, _D), lambda i: (0, 0)),
                  pl.BlockSpec((1, _OUT), lambda i: (0, 0))],
        out_specs=pl.BlockSpec((_BR, _OUT), lambda i: (i, 0)),
        out_shape=jax.ShapeDtypeStruct((_N, _OUT), jnp.float32),
    )(h, Wh, bh.reshape(1, _OUT))


def _prep_edges(src, dst):
    pad = _EPAD - _E
    srcp = jnp.concatenate([src, jnp.zeros((pad,), jnp.int32)])
    dstp = jnp.concatenate([dst, jnp.full((pad,), _N, jnp.int32)])
    s4 = (srcp[None, :] * 4
          + jnp.arange(4, dtype=jnp.int32)[:, None]).reshape(4, _CHUNKS, _CH)
    return s4, dstp.reshape(_CHUNKS, _CH)


def kernel(x_user, x_item, edge_index_ui, edge_index_iu, Wp_user, bp_user,
           Wp_item, bp_item, Wl0, bl0, Wr0, gamma0, beta0, Wl1, bl1, Wr1,
           gamma1, beta1, Wh, bh):
    ei_ui = edge_index_ui.astype(jnp.int32)
    ei_iu = edge_index_iu.astype(jnp.int32)
    s4_iu, d_iu = _prep_edges(ei_iu[0], ei_iu[1])
    s4_ui, d_ui = _prep_edges(ei_ui[0], ei_ui[1])
    dcat = jnp.stack([d_iu, d_ui])

    ones16 = jnp.ones((_CH, 16), jnp.float32)
    z16 = jnp.zeros((_RPT, 16), jnp.float32)
    zb = jnp.zeros((_RPT, 32), jnp.bfloat16)

    cnt = _sc_count(dcat, ones16, z16)
    cnt_u, cnt_i = cnt[0], cnt[1]

    hu, hub = _proj(x_user, Wp_user, bp_user)
    hi, hib = _proj(x_item, Wp_item, bp_item)

    inv = 1.0 / jnp.sqrt(1.0 + _EPS)
    one = jnp.ones((_D,), jnp.float32)
    zero = jnp.zeros((_D,), jnp.float32)

    for li, (Wl, bl, Wr, gm, bt) in enumerate(
            ((Wl0, bl0, Wr0, gamma0, beta0), (Wl1, bl1, Wr1, gamma1, beta1))):
        ssum_u = _sc_seg(hib.reshape(_N * 4, 32), s4_iu, d_iu, zb)
        ssum_i = _sc_seg(hub.reshape(_N * 4, 32), s4_ui, d_ui, zb)
        if li == 0:
            hu, hub = _layer(hu, ssum_u, cnt_u, Wl, bl, Wr, gm * inv, bt)
            hi, hib = _layer(hi, ssum_i, cnt_i, Wl, bl, Wr, one, zero)
        else:
            hu = _layer(hu, ssum_u, cnt_u, Wl, bl, Wr, gm * inv, bt,
                        want_bf16=False)
            hi = _layer(hi, ssum_i, cnt_i, Wl, bl, Wr, one, zero,
                        want_bf16=False)

    return _head(hu, Wh, bh), hu


# speedup vs baseline: 1.3685x; 1.0163x over previous
import functools

import jax
import jax.numpy as jnp
from jax import lax
from jax.experimental import pallas as pl
from jax.experimental.pallas import tpu as pltpu
from jax.experimental.pallas import tpu_sc as plsc

_N = 50000
_E = 300000
_D = 128
_OUT = 64
_EPS = 1e-5

_NC, _NS = 2, 16
_CH = 128
_CHUNKS = -(-_E // (_NS * _CH)) * _NS
_EPAD = _CHUNKS * _CH
_CPT = _CHUNKS // _NS
_K = 7
_NG = _CPT // _K
_NPAD = 50048
_RPT = _NPAD // _NS

_BR = 512
_GRID = -(-_N // _BR)

_mesh = plsc.VectorSubcoreMesh(
    core_axis_name="c", subcore_axis_name="s", num_cores=_NC, num_subcores=_NS
)
_sc_params = pltpu.CompilerParams(use_tc_tiling_on_sc=False)


@functools.partial(
    pl.kernel,
    out_type=jax.ShapeDtypeStruct((2, _NPAD, _D), jnp.float32),
    mesh=_mesh,
    scratch_types=[
        pltpu.VMEM_SHARED((_NPAD, 16), jnp.float32),
        pltpu.VMEM((_CH, 16), jnp.float32),
        pltpu.VMEM((_CPT, _CH), jnp.int32),
        pltpu.SemaphoreType.DMA,
    ],
    compiler_params=_sc_params,
)
def _sc_count(dcat, ones_h, z16, cnt_out, acc, ones_v, didx, sem):
    c = lax.axis_index("c")
    s = lax.axis_index("s")
    pltpu.sync_copy(ones_h, ones_v)
    pltpu.sync_copy(dcat.at[c, pl.ds(s * _CPT, _CPT)], didx)
    pltpu.sync_copy(z16, acc.at[pl.ds(s * _RPT, _RPT)])
    plsc.subcore_barrier()

    def body(t, carry):
        for b in range(_K):
            pltpu.async_copy(ones_v, acc.at[didx.at[t * _K + b]], sem,
                             add=True)
        for b in range(_K):
            pltpu.make_async_copy(z16.at[pl.ds(0, _CH)], ones_v, sem).wait()
        return carry

    lax.fori_loop(0, _NG, body, 0)
    plsc.subcore_barrier()
    pltpu.sync_copy(acc.at[pl.ds(s * _RPT, _RPT)],
                    cnt_out.at[c, pl.ds(s * _RPT, _RPT), pl.ds(0, 16)])


@functools.partial(
    pl.kernel,
    out_type=jax.ShapeDtypeStruct((_NPAD, _D), jnp.bfloat16),
    mesh=_mesh,
    scratch_types=[
        pltpu.VMEM_SHARED((_NPAD, 32), jnp.bfloat16),
        pltpu.VMEM((_RPT // 4, 32), jnp.bfloat16),
        pltpu.VMEM((_CPT, _CH), jnp.int32),
        pltpu.VMEM((_CPT, _CH), jnp.int32),
        [pltpu.VMEM((_CH, 32), jnp.bfloat16)] * (2 * _K),
        [pltpu.SemaphoreType.DMA] * 4,
    ],
    compiler_params=_sc_params,
)
def _sc_seg(table, s4, dd, zb, out,
            acc, zbuf, sidx, didx, rows, sems):
    c = lax.axis_index("c")
    s = lax.axis_index("s")
    rows_ab = (rows[:_K], rows[_K:])
    gsem = (sems[0], sems[1])
    ssem = (sems[2], sems[3])
    pltpu.sync_copy(zb.at[pl.ds(0, _RPT // 4)], zbuf)

    def fire_g(table, g, ab):
        for b in range(_K):
            pltpu.async_copy(table.at[sidx.at[g * _K + b]], rows_ab[ab][b],
                             gsem[ab])

    def drain_g(ab):
        for b in range(_K):
            pltpu.make_async_copy(zb.at[pl.ds(0, _CH)], rows_ab[ab][b],
                                  gsem[ab]).wait()

    def fire_s(g, ab):
        for b in range(_K):
            pltpu.async_copy(rows_ab[ab][b], acc.at[didx.at[g * _K + b]],
                             ssem[ab], add=True)

    def drain_s(ab):
        for b in range(_K):
            pltpu.make_async_copy(zb.at[pl.ds(0, _CH)], rows_ab[ab][b],
                                  ssem[ab]).wait()

    if True:
        pltpu.sync_copy(dd.at[pl.ds(s * _CPT, _CPT)], didx)
        for p_loc in range(2):
            p = c * 2 + p_loc
            pltpu.sync_copy(s4.at[p, pl.ds(s * _CPT, _CPT)], sidx)
            fire_g(table, 0, 0)
            fire_g(table, 1, 1)
            for z in range(4):
                pltpu.sync_copy(
                    zbuf,
                    acc.at[pl.ds(s * _RPT + z * (_RPT // 4), _RPT // 4)])
            plsc.subcore_barrier()

            drain_g(0)
            fire_s(0, 0)

            def pair(t, carry):
                for par in range(2):
                    g = 2 * t + 1 + par
                    ab = (1, 0)[par]
                    drain_g(ab)
                    fire_s(g, ab)
                    drain_s(1 - ab)
                    fire_g(table, g + 1, 1 - ab)
                return carry

            lax.fori_loop(0, (_NG - 3) // 2, pair, 0)
            drain_g(1)
            fire_s(_NG - 2, 1)
            drain_s(0)
            fire_g(table, _NG - 1, 0)
            drain_g(0)
            fire_s(_NG - 1, 0)
            drain_s(1)
            drain_s(0)
            plsc.subcore_barrier()
            pltpu.sync_copy(acc.at[pl.ds(s * _RPT, _RPT)],
                            out.at[pl.ds(s * _RPT, _RPT), pl.ds(p * 32, 32)])


def _proj_body(x_ref, w_ref, b_ref, o_ref, ob_ref):
    y = lax.dot_general(x_ref[...], w_ref[...], (((1,), (1,)), ((), ())),
                        preferred_element_type=jnp.float32)
    y = jnp.maximum(y + b_ref[...], 0.0)
    o_ref[...] = y
    ob_ref[...] = y.astype(jnp.bfloat16)


def _proj(x, W, b):
    return pl.pallas_call(
        _proj_body,
        grid=(_GRID,),
        in_specs=[pl.BlockSpec((_BR, _D), lambda i: (i, 0)),
                  pl.BlockSpec((_D, _D), lambda i: (0, 0)),
                  pl.BlockSpec((1, _D), lambda i: (0, 0))],
        out_specs=(pl.BlockSpec((_BR, _D), lambda i: (i, 0)),
                   pl.BlockSpec((_BR, _D), lambda i: (i, 0))),
        out_shape=(jax.ShapeDtypeStruct((_N, _D), jnp.float32),
                   jax.ShapeDtypeStruct((_N, _D), jnp.bfloat16)),
    )(x, W, b.reshape(1, _D))


def _layer_body(h_ref, ss_ref, cnt_ref, wl_ref, bl_ref, wr_ref, sc_ref,
                sh_ref, o_ref, ob_ref=None):
    h = h_ref[...]
    mean = (ss_ref[...].astype(jnp.float32)
            / jnp.maximum(cnt_ref[...][:, :1], 1.0))
    agg = lax.dot_general(mean, wl_ref[...], (((1,), (1,)), ((), ())),
                          preferred_element_type=jnp.float32)
    agg += lax.dot_general(h, wr_ref[...], (((1,), (1,)), ((), ())),
                           preferred_element_type=jnp.float32)
    t = (h + agg + bl_ref[...]) * 0.5
    y = jnp.maximum(t * sc_ref[...] + sh_ref[...], 0.0)
    o_ref[...] = y
    if ob_ref is not None:
        ob_ref[...] = y.astype(jnp.bfloat16)


def _layer(h, ss4, cnt, Wl, bl, Wr, scale, shift, want_bf16=True):
    if want_bf16:
        out_specs = (pl.BlockSpec((_BR, _D), lambda i: (i, 0)),
                     pl.BlockSpec((_BR, _D), lambda i: (i, 0)))
        out_shape = (jax.ShapeDtypeStruct((_N, _D), jnp.float32),
                     jax.ShapeDtypeStruct((_N, _D), jnp.bfloat16))
    else:
        out_specs = pl.BlockSpec((_BR, _D), lambda i: (i, 0))
        out_shape = jax.ShapeDtypeStruct((_N, _D), jnp.float32)
    return pl.pallas_call(
        _layer_body,
        grid=(_GRID,),
        in_specs=[pl.BlockSpec((_BR, _D), lambda i: (i, 0)),
                  pl.BlockSpec((_BR, _D), lambda i: (i, 0)),
                  pl.BlockSpec((_BR, _D), lambda i: (i, 0)),
                  pl.BlockSpec((_D, _D), lambda i: (0, 0)),
                  pl.BlockSpec((1, _D), lambda i: (0, 0)),
                  pl.BlockSpec((_D, _D), lambda i: (0, 0)),
                  pl.BlockSpec((1, _D), lambda i: (0, 0)),
                  pl.BlockSpec((1, _D), lambda i: (0, 0))],
        out_specs=out_specs,
        out_shape=out_shape,
    )(h, ss4, cnt, Wl, bl.reshape(1, _D), Wr, scale.reshape(1, _D),
      shift.reshape(1, _D))


def _head_body(h_ref, w_ref, b_ref, o_ref):
    y = lax.dot_general(h_ref[...], w_ref[...], (((1,), (1,)), ((), ())),
                        preferred_element_type=jnp.float32)
    o_ref[...] = y + b_ref[...]


def _head(h, Wh, bh):
    return pl.pallas_call(
        _head_body,
        grid=(_GRID,),
        in_specs=[pl.BlockSpec((_BR, _D), lambda i: (i, 0)),
                  pl.BlockSpec((_OUT, _D), lambda i: (0, 0)),
                  pl.BlockSpec((1, _OUT), lambda i: (0, 0))],
        out_specs=pl.BlockSpec((_BR, _OUT), lambda i: (i, 0)),
        out_shape=jax.ShapeDtypeStruct((_N, _OUT), jnp.float32),
    )(h, Wh, bh.reshape(1, _OUT))


def _prep_edges(src, dst):
    pad = _EPAD - _E
    srcp = jnp.concatenate([src, jnp.zeros((pad,), jnp.int32)])
    dstp = jnp.concatenate([dst, jnp.full((pad,), _N, jnp.int32)])
    s4 = (srcp[None, :] * 4
          + jnp.arange(4, dtype=jnp.int32)[:, None]).reshape(4, _CHUNKS, _CH)
    return s4, dstp.reshape(_CHUNKS, _CH)


def kernel(x_user, x_item, edge_index_ui, edge_index_iu, Wp_user, bp_user,
           Wp_item, bp_item, Wl0, bl0, Wr0, gamma0, beta0, Wl1, bl1, Wr1,
           gamma1, beta1, Wh, bh):
    ei_ui = edge_index_ui.astype(jnp.int32)
    ei_iu = edge_index_iu.astype(jnp.int32)
    s4_iu, d_iu = _prep_edges(ei_iu[0], ei_iu[1])
    s4_ui, d_ui = _prep_edges(ei_ui[0], ei_ui[1])
    dcat = jnp.stack([d_iu, d_ui])

    ones16 = jnp.ones((_CH, 16), jnp.float32)
    z16 = jnp.zeros((_RPT, 16), jnp.float32)
    zb = jnp.zeros((_RPT, 32), jnp.bfloat16)

    cnt = _sc_count(dcat, ones16, z16)
    cnt_u, cnt_i = cnt[0], cnt[1]

    hu, hub = _proj(x_user, Wp_user, bp_user)
    hi, hib = _proj(x_item, Wp_item, bp_item)

    inv = 1.0 / jnp.sqrt(1.0 + _EPS)
    one = jnp.ones((_D,), jnp.float32)
    zero = jnp.zeros((_D,), jnp.float32)

    for li, (Wl, bl, Wr, gm, bt) in enumerate(
            ((Wl0, bl0, Wr0, gamma0, beta0), (Wl1, bl1, Wr1, gamma1, beta1))):
        ssum_u = _sc_seg(hib.reshape(_N * 4, 32), s4_iu, d_iu, zb)
        ssum_i = _sc_seg(hub.reshape(_N * 4, 32), s4_ui, d_ui, zb)
        if li == 0:
            hu, hub = _layer(hu, ssum_u, cnt_u, Wl, bl, Wr, gm * inv, bt)
            hi, hib = _layer(hi, ssum_i, cnt_i, Wl, bl, Wr, one, zero)
        else:
            hu = _layer(hu, ssum_u, cnt_u, Wl, bl, Wr, gm * inv, bt,
                        want_bf16=False)
            hi = _layer(hi, ssum_i, cnt_i, Wl, bl, Wr, one, zero,
                        want_bf16=False)

    return _head(hu, Wh, bh), hu


# speedup vs baseline: 1.3688x; 1.0002x over previous
import functools

import jax
import jax.numpy as jnp
from jax import lax
from jax.experimental import pallas as pl
from jax.experimental.pallas import tpu as pltpu
from jax.experimental.pallas import tpu_sc as plsc

_N = 50000
_E = 300000
_D = 128
_OUT = 64
_EPS = 1e-5

_NC, _NS = 2, 16
_CH = 128
_CHUNKS = -(-_E // (_NS * _CH)) * _NS
_EPAD = _CHUNKS * _CH
_CPT = _CHUNKS // _NS
_K = 7
_NG = _CPT // _K
_NPAD = 50048
_RPT = _NPAD // _NS

_BR = 512
_GRID = -(-_N // _BR)

_mesh = plsc.VectorSubcoreMesh(
    core_axis_name="c", subcore_axis_name="s", num_cores=_NC, num_subcores=_NS
)
_sc_params = pltpu.CompilerParams(use_tc_tiling_on_sc=False)


@functools.partial(
    pl.kernel,
    out_type=jax.ShapeDtypeStruct((2, _NPAD, _D), jnp.float32),
    mesh=_mesh,
    scratch_types=[
        pltpu.VMEM_SHARED((_NPAD, 16), jnp.float32),
        pltpu.VMEM((_CH, 16), jnp.float32),
        pltpu.VMEM((_CPT, _CH), jnp.int32),
        pltpu.SemaphoreType.DMA,
    ],
    compiler_params=_sc_params,
)
def _sc_count(dcat, ones_h, z16, cnt_out, acc, ones_v, didx, sem):
    c = lax.axis_index("c")
    s = lax.axis_index("s")
    pltpu.sync_copy(ones_h, ones_v)
    pltpu.sync_copy(dcat.at[c, pl.ds(s * _CPT, _CPT)], didx)
    pltpu.sync_copy(z16, acc.at[pl.ds(s * _RPT, _RPT)])
    plsc.subcore_barrier()

    def body(t, carry):
        for b in range(_K):
            pltpu.async_copy(ones_v, acc.at[didx.at[t * _K + b]], sem,
                             add=True)
        for b in range(_K):
            pltpu.make_async_copy(z16.at[pl.ds(0, _CH)], ones_v, sem).wait()
        return carry

    lax.fori_loop(0, _NG, body, 0)
    plsc.subcore_barrier()
    pltpu.sync_copy(acc.at[pl.ds(s * _RPT, _RPT)],
                    cnt_out.at[c, pl.ds(s * _RPT, _RPT), pl.ds(0, 16)])


@functools.partial(
    pl.kernel,
    out_type=jax.ShapeDtypeStruct((_NPAD, _D), jnp.bfloat16),
    mesh=_mesh,
    scratch_types=[
        pltpu.VMEM_SHARED((_NPAD, 32), jnp.bfloat16),
        pltpu.VMEM((_RPT // 4, 32), jnp.bfloat16),
        pltpu.VMEM((_CPT, _CH), jnp.int32),
        pltpu.VMEM((_CPT, _CH), jnp.int32),
        [pltpu.VMEM((_CH, 32), jnp.bfloat16)] * (2 * _K),
        [pltpu.SemaphoreType.DMA] * 4,
    ],
    compiler_params=_sc_params,
)
def _sc_seg(table, s4, dd, zb, out,
            acc, zbuf, sidx, didx, rows, sems):
    c = lax.axis_index("c")
    s = lax.axis_index("s")
    rows_ab = (rows[:_K], rows[_K:])
    gsem = (sems[0], sems[1])
    ssem = (sems[2], sems[3])
    pltpu.sync_copy(zb.at[pl.ds(0, _RPT // 4)], zbuf)

    def fire_g(table, g, ab):
        for b in range(_K):
            pltpu.async_copy(table.at[sidx.at[g * _K + b]], rows_ab[ab][b],
                             gsem[ab])

    def drain_g(ab):
        for b in range(_K):
            pltpu.make_async_copy(zb.at[pl.ds(0, _CH)], rows_ab[ab][b],
                                  gsem[ab]).wait()

    def fire_s(g, ab):
        for b in range(_K):
            pltpu.async_copy(rows_ab[ab][b], acc.at[didx.at[g * _K + b]],
                             ssem[ab], add=True)

    def drain_s(ab):
        for b in range(_K):
            pltpu.make_async_copy(zb.at[pl.ds(0, _CH)], rows_ab[ab][b],
                                  ssem[ab]).wait()

    pltpu.sync_copy(dd.at[pl.ds(s * _CPT, _CPT)], didx)
    for p_loc in range(2):
        p = c * 2 + p_loc
        pltpu.sync_copy(s4.at[p, pl.ds(s * _CPT, _CPT)], sidx)
        fire_g(table, 0, 0)
        fire_g(table, 1, 1)
        for z in range(4):
            pltpu.sync_copy(
                zbuf,
                acc.at[pl.ds(s * _RPT + z * (_RPT // 4), _RPT // 4)])
        plsc.subcore_barrier()

        drain_g(0)
        fire_s(0, 0)

        def pair(t, carry):
            for par in range(2):
                g = 2 * t + 1 + par
                ab = (1, 0)[par]
                drain_g(ab)
                fire_s(g, ab)
                drain_s(1 - ab)
                fire_g(table, g + 1, 1 - ab)
            return carry

        lax.fori_loop(0, (_NG - 3) // 2, pair, 0)
        drain_g(1)
        fire_s(_NG - 2, 1)
        drain_s(0)
        fire_g(table, _NG - 1, 0)
        drain_g(0)
        fire_s(_NG - 1, 0)
        drain_s(1)
        drain_s(0)
        plsc.subcore_barrier()
        pltpu.sync_copy(acc.at[pl.ds(s * _RPT, _RPT)],
                        out.at[pl.ds(s * _RPT, _RPT), pl.ds(p * 32, 32)])


def _proj_body(x_ref, w_ref, b_ref, o_ref, ob_ref):
    y = lax.dot_general(x_ref[...], w_ref[...], (((1,), (1,)), ((), ())),
                        preferred_element_type=jnp.float32)
    y = jnp.maximum(y + b_ref[...], 0.0)
    o_ref[...] = y
    ob_ref[...] = y.astype(jnp.bfloat16)


def _proj(x, W, b):
    return pl.pallas_call(
        _proj_body,
        grid=(_GRID,),
        in_specs=[pl.BlockSpec((_BR, _D), lambda i: (i, 0)),
                  pl.BlockSpec((_D, _D), lambda i: (0, 0)),
                  pl.BlockSpec((1, _D), lambda i: (0, 0))],
        out_specs=(pl.BlockSpec((_BR, _D), lambda i: (i, 0)),
                   pl.BlockSpec((_BR, _D), lambda i: (i, 0))),
        out_shape=(jax.ShapeDtypeStruct((_N, _D), jnp.float32),
                   jax.ShapeDtypeStruct((_N, _D), jnp.bfloat16)),
    )(x, W, b.reshape(1, _D))


def _layer_body(h_ref, ss_ref, cnt_ref, wl_ref, bl_ref, wr_ref, sc_ref,
                sh_ref, o_ref, ob_ref=None):
    h = h_ref[...]
    mean = (ss_ref[...].astype(jnp.float32)
            / jnp.maximum(cnt_ref[...][:, :1], 1.0))
    agg = lax.dot_general(mean, wl_ref[...], (((1,), (1,)), ((), ())),
                          preferred_element_type=jnp.float32)
    agg += lax.dot_general(h, wr_ref[...], (((1,), (1,)), ((), ())),
                           preferred_element_type=jnp.float32)
    t = (h + agg + bl_ref[...]) * 0.5
    y = jnp.maximum(t * sc_ref[...] + sh_ref[...], 0.0)
    o_ref[...] = y
    if ob_ref is not None:
        ob_ref[...] = y.astype(jnp.bfloat16)


def _layer(h, ss4, cnt, Wl, bl, Wr, scale, shift, want_bf16=True):
    if want_bf16:
        out_specs = (pl.BlockSpec((_BR, _D), lambda i: (i, 0)),
                     pl.BlockSpec((_BR, _D), lambda i: (i, 0)))
        out_shape = (jax.ShapeDtypeStruct((_N, _D), jnp.float32),
                     jax.ShapeDtypeStruct((_N, _D), jnp.bfloat16))
    else:
        out_specs = pl.BlockSpec((_BR, _D), lambda i: (i, 0))
        out_shape = jax.ShapeDtypeStruct((_N, _D), jnp.float32)
    return pl.pallas_call(
        _layer_body,
        grid=(_GRID,),
        in_specs=[pl.BlockSpec((_BR, _D), lambda i: (i, 0)),
                  pl.BlockSpec((_BR, _D), lambda i: (i, 0)),
                  pl.BlockSpec((_BR, _D), lambda i: (i, 0)),
                  pl.BlockSpec((_D, _D), lambda i: (0, 0)),
                  pl.BlockSpec((1, _D), lambda i: (0, 0)),
                  pl.BlockSpec((_D, _D), lambda i: (0, 0)),
                  pl.BlockSpec((1, _D), lambda i: (0, 0)),
                  pl.BlockSpec((1, _D), lambda i: (0, 0))],
        out_specs=out_specs,
        out_shape=out_shape,
    )(h, ss4, cnt, Wl, bl.reshape(1, _D), Wr, scale.reshape(1, _D),
      shift.reshape(1, _D))


def _head_body(h_ref, w_ref, b_ref, o_ref):
    y = lax.dot_general(h_ref[...], w_ref[...], (((1,), (1,)), ((), ())),
                        preferred_element_type=jnp.float32)
    o_ref[...] = y + b_ref[...]


def _head(h, Wh, bh):
    return pl.pallas_call(
        _head_body,
        grid=(_GRID,),
        in_specs=[pl.BlockSpec((_BR, _D), lambda i: (i, 0)),
                  pl.BlockSpec((_OUT, _D), lambda i: (0, 0)),
                  pl.BlockSpec((1, _OUT), lambda i: (0, 0))],
        out_specs=pl.BlockSpec((_BR, _OUT), lambda i: (i, 0)),
        out_shape=jax.ShapeDtypeStruct((_N, _OUT), jnp.float32),
    )(h, Wh, bh.reshape(1, _OUT))


def _prep_edges(src, dst):
    pad = _EPAD - _E
    srcp = jnp.concatenate([src, jnp.zeros((pad,), jnp.int32)])
    dstp = jnp.concatenate([dst, jnp.full((pad,), _N, jnp.int32)])
    s4 = (srcp[None, :] * 4
          + jnp.arange(4, dtype=jnp.int32)[:, None]).reshape(4, _CHUNKS, _CH)
    return s4, dstp.reshape(_CHUNKS, _CH)


def kernel(x_user, x_item, edge_index_ui, edge_index_iu, Wp_user, bp_user,
           Wp_item, bp_item, Wl0, bl0, Wr0, gamma0, beta0, Wl1, bl1, Wr1,
           gamma1, beta1, Wh, bh):
    ei_ui = edge_index_ui.astype(jnp.int32)
    ei_iu = edge_index_iu.astype(jnp.int32)
    s4_iu, d_iu = _prep_edges(ei_iu[0], ei_iu[1])
    s4_ui, d_ui = _prep_edges(ei_ui[0], ei_ui[1])
    dcat = jnp.stack([d_iu, d_ui])

    ones16 = jnp.ones((_CH, 16), jnp.float32)
    z16 = jnp.zeros((_RPT, 16), jnp.float32)
    zb = jnp.zeros((_RPT, 32), jnp.bfloat16)

    cnt = _sc_count(dcat, ones16, z16)
    cnt_u, cnt_i = cnt[0], cnt[1]

    hu, hub = _proj(x_user, Wp_user, bp_user)
    hi, hib = _proj(x_item, Wp_item, bp_item)

    inv = 1.0 / jnp.sqrt(1.0 + _EPS)
    one = jnp.ones((_D,), jnp.float32)
    zero = jnp.zeros((_D,), jnp.float32)

    for li, (Wl, bl, Wr, gm, bt) in enumerate(
            ((Wl0, bl0, Wr0, gamma0, beta0), (Wl1, bl1, Wr1, gamma1, beta1))):
        ssum_u = _sc_seg(hib.reshape(_N * 4, 32), s4_iu, d_iu, zb)
        ssum_i = _sc_seg(hub.reshape(_N * 4, 32), s4_ui, d_ui, zb)
        if li == 0:
            hu, hub = _layer(hu, ssum_u, cnt_u, Wl, bl, Wr, gm * inv, bt)
            hi, hib = _layer(hi, ssum_i, cnt_i, Wl, bl, Wr, one, zero)
        else:
            hu = _layer(hu, ssum_u, cnt_u, Wl, bl, Wr, gm * inv, bt,
                        want_bf16=False)
            hi = _layer(hi, ssum_i, cnt_i, Wl, bl, Wr, one, zero,
                        want_bf16=False)

    return _head(hu, Wh, bh), hu


# speedup vs baseline: 1.4397x; 1.0518x over previous
import functools

import jax
import jax.numpy as jnp
from jax import lax
from jax.experimental import pallas as pl
from jax.experimental.pallas import tpu as pltpu
from jax.experimental.pallas import tpu_sc as plsc

_N = 50000
_E = 300000
_D = 128
_OUT = 64
_EPS = 1e-5

_NC, _NS = 2, 16
_CH = 128
_CHUNKS = -(-_E // (_NS * _CH)) * _NS
_EPAD = _CHUNKS * _CH
_CPT = _CHUNKS // _NS
_K = 7
_NG = _CPT // _K
_NPAD = 50048
_RPT = _NPAD // _NS

_BR = 512
_GRID = -(-_N // _BR)

_mesh = plsc.VectorSubcoreMesh(
    core_axis_name="c", subcore_axis_name="s", num_cores=_NC, num_subcores=_NS
)
_sc_params = pltpu.CompilerParams(use_tc_tiling_on_sc=False)


@functools.partial(
    pl.kernel,
    out_type=jax.ShapeDtypeStruct((2, _NPAD, _D), jnp.float32),
    mesh=_mesh,
    scratch_types=[
        pltpu.VMEM_SHARED((_NPAD, 16), jnp.float32),
        pltpu.VMEM((_CH, 16), jnp.float32),
        pltpu.VMEM((_CPT, _CH), jnp.int32),
        pltpu.SemaphoreType.DMA,
    ],
    compiler_params=_sc_params,
)
def _sc_count(dcat, ones_h, z16, cnt_out, acc, ones_v, didx, sem):
    c = lax.axis_index("c")
    s = lax.axis_index("s")
    pltpu.sync_copy(ones_h, ones_v)
    pltpu.sync_copy(dcat.at[c, pl.ds(s * _CPT, _CPT)], didx)
    pltpu.sync_copy(z16, acc.at[pl.ds(s * _RPT, _RPT)])
    plsc.subcore_barrier()

    def body(t, carry):
        for b in range(_K):
            pltpu.async_copy(ones_v, acc.at[didx.at[t * _K + b]], sem,
                             add=True)
        for b in range(_K):
            pltpu.make_async_copy(z16.at[pl.ds(0, _CH)], ones_v, sem).wait()
        return carry

    lax.fori_loop(0, _NG, body, 0)
    plsc.subcore_barrier()
    pltpu.sync_copy(acc.at[pl.ds(s * _RPT, _RPT)],
                    cnt_out.at[c, pl.ds(s * _RPT, _RPT), pl.ds(0, 16)])


@functools.partial(
    pl.kernel,
    out_type=jax.ShapeDtypeStruct((_NPAD, _D), jnp.bfloat16),
    mesh=_mesh,
    scratch_types=[
        pltpu.VMEM_SHARED((_NPAD, 32), jnp.bfloat16),
        pltpu.VMEM((_RPT // 4, 32), jnp.bfloat16),
        pltpu.VMEM((_CPT, _CH), jnp.int32),
        pltpu.VMEM((_CPT, _CH), jnp.int32),
        [pltpu.VMEM((_CH, 32), jnp.bfloat16)] * (2 * _K),
        [pltpu.SemaphoreType.DMA] * 4,
    ],
    compiler_params=_sc_params,
)
def _sc_seg(table, s4, dd, zb, out,
            acc, zbuf, sidx, didx, rows, sems):
    c = lax.axis_index("c")
    s = lax.axis_index("s")
    rows_ab = (rows[:_K], rows[_K:])
    gsem = (sems[0], sems[1])
    ssem = (sems[2], sems[3])
    pltpu.sync_copy(zb.at[pl.ds(0, _RPT // 4)], zbuf)

    def fire_g(table, g, ab):
        for b in range(_K):
            pltpu.async_copy(table.at[sidx.at[g * _K + b]], rows_ab[ab][b],
                             gsem[ab])

    def drain_g(ab):
        for b in range(_K):
            pltpu.make_async_copy(zb.at[pl.ds(0, _CH)], rows_ab[ab][b],
                                  gsem[ab]).wait()

    def fire_s(g, ab):
        for b in range(_K):
            pltpu.async_copy(rows_ab[ab][b], acc.at[didx.at[g * _K + b]],
                             ssem[ab], add=True)

    def drain_s(ab):
        for b in range(_K):
            pltpu.make_async_copy(zb.at[pl.ds(0, _CH)], rows_ab[ab][b],
                                  ssem[ab]).wait()

    pltpu.sync_copy(dd.at[pl.ds(s * _CPT, _CPT)], didx)
    for p_loc in range(2):
        p = c * 2 + p_loc
        pltpu.sync_copy(s4.at[p, pl.ds(s * _CPT, _CPT)], sidx)
        fire_g(table, 0, 0)
        fire_g(table, 1, 1)
        for z in range(4):
            pltpu.sync_copy(
                zbuf,
                acc.at[pl.ds(s * _RPT + z * (_RPT // 4), _RPT // 4)])
        plsc.subcore_barrier()

        drain_g(0)
        fire_s(0, 0)

        def pair(t, carry):
            for par in range(2):
                g = 2 * t + 1 + par
                ab = (1, 0)[par]
                drain_g(ab)
                fire_s(g, ab)
                drain_s(1 - ab)
                fire_g(table, g + 1, 1 - ab)
            return carry

        lax.fori_loop(0, (_NG - 3) // 2, pair, 0)
        drain_g(1)
        fire_s(_NG - 2, 1)
        drain_s(0)
        fire_g(table, _NG - 1, 0)
        drain_g(0)
        fire_s(_NG - 1, 0)
        drain_s(1)
        drain_s(0)
        plsc.subcore_barrier()
        pltpu.sync_copy(acc.at[pl.ds(s * _RPT, _RPT)],
                        out.at[pl.ds(s * _RPT, _RPT), pl.ds(p * 32, 32)])


def _proj_body(x_ref, w_ref, b_ref, o_ref, ob_ref):
    y = lax.dot_general(x_ref[...], w_ref[...], (((1,), (1,)), ((), ())),
                        preferred_element_type=jnp.float32)
    y = jnp.maximum(y + b_ref[...], 0.0)
    o_ref[...] = y
    ob_ref[...] = y.astype(jnp.bfloat16)


def _proj(x, W, b):
    return pl.pallas_call(
        _proj_body,
        grid=(_GRID,),
        in_specs=[pl.BlockSpec((_BR, _D), lambda i: (i, 0)),
                  pl.BlockSpec((_D, _D), lambda i: (0, 0)),
                  pl.BlockSpec((1, _D), lambda i: (0, 0))],
        out_specs=(pl.BlockSpec((_BR, _D), lambda i: (i, 0)),
                   pl.BlockSpec((_BR, _D), lambda i: (i, 0))),
        out_shape=(jax.ShapeDtypeStruct((_N, _D), jnp.float32),
                   jax.ShapeDtypeStruct((_N, _D), jnp.bfloat16)),
    )(x, W, b.reshape(1, _D))


def _layer_body(h_ref, ss_ref, cnt_ref, wl_ref, bl_ref, wr_ref, sc_ref,
                sh_ref, o_ref, ob_ref=None):
    h = h_ref[...]
    mean = (ss_ref[...].astype(jnp.float32)
            / jnp.maximum(cnt_ref[...][:, :1], 1.0))
    agg = lax.dot_general(mean, wl_ref[...], (((1,), (1,)), ((), ())),
                          preferred_element_type=jnp.float32)
    agg += lax.dot_general(h, wr_ref[...], (((1,), (1,)), ((), ())),
                           preferred_element_type=jnp.float32)
    t = (h + agg + bl_ref[...]) * 0.5
    y = jnp.maximum(t * sc_ref[...] + sh_ref[...], 0.0)
    o_ref[...] = y
    if ob_ref is not None:
        ob_ref[...] = y.astype(jnp.bfloat16)


def _layer(h, ss4, cnt, Wl, bl, Wr, scale, shift, want_bf16=True):
    if want_bf16:
        out_specs = (pl.BlockSpec((_BR, _D), lambda i: (i, 0)),
                     pl.BlockSpec((_BR, _D), lambda i: (i, 0)))
        out_shape = (jax.ShapeDtypeStruct((_N, _D), jnp.float32),
                     jax.ShapeDtypeStruct((_N, _D), jnp.bfloat16))
    else:
        out_specs = pl.BlockSpec((_BR, _D), lambda i: (i, 0))
        out_shape = jax.ShapeDtypeStruct((_N, _D), jnp.float32)
    return pl.pallas_call(
        _layer_body,
        grid=(_GRID,),
        in_specs=[pl.BlockSpec((_BR, _D), lambda i: (i, 0)),
                  pl.BlockSpec((_BR, _D), lambda i: (i, 0)),
                  pl.BlockSpec((_BR, _D), lambda i: (i, 0)),
                  pl.BlockSpec((_D, _D), lambda i: (0, 0)),
                  pl.BlockSpec((1, _D), lambda i: (0, 0)),
                  pl.BlockSpec((_D, _D), lambda i: (0, 0)),
                  pl.BlockSpec((1, _D), lambda i: (0, 0)),
                  pl.BlockSpec((1, _D), lambda i: (0, 0))],
        out_specs=out_specs,
        out_shape=out_shape,
    )(h, ss4, cnt, Wl, bl.reshape(1, _D), Wr, scale.reshape(1, _D),
      shift.reshape(1, _D))


def _layer_head_body(h_ref, ss_ref, cnt_ref, wl_ref, bl_ref, wr_ref,
                     sc_ref, sh_ref, wh_ref, bh_ref, o_ref, oo_ref):
    h = h_ref[...]
    mean = (ss_ref[...].astype(jnp.float32)
            / jnp.maximum(cnt_ref[...][:, :1], 1.0))
    agg = lax.dot_general(mean, wl_ref[...], (((1,), (1,)), ((), ())),
                          preferred_element_type=jnp.float32)
    agg += lax.dot_general(h, wr_ref[...], (((1,), (1,)), ((), ())),
                           preferred_element_type=jnp.float32)
    t = (h + agg + bl_ref[...]) * 0.5
    y = jnp.maximum(t * sc_ref[...] + sh_ref[...], 0.0)
    o_ref[...] = y
    oo_ref[...] = lax.dot_general(
        y, wh_ref[...], (((1,), (1,)), ((), ())),
        preferred_element_type=jnp.float32) + bh_ref[...]


def _layer_head(h, ss4, cnt, Wl, bl, Wr, scale, shift, Wh, bh):
    return pl.pallas_call(
        _layer_head_body,
        grid=(_GRID,),
        in_specs=[pl.BlockSpec((_BR, _D), lambda i: (i, 0)),
                  pl.BlockSpec((_BR, _D), lambda i: (i, 0)),
                  pl.BlockSpec((_BR, _D), lambda i: (i, 0)),
                  pl.BlockSpec((_D, _D), lambda i: (0, 0)),
                  pl.BlockSpec((1, _D), lambda i: (0, 0)),
                  pl.BlockSpec((_D, _D), lambda i: (0, 0)),
                  pl.BlockSpec((1, _D), lambda i: (0, 0)),
                  pl.BlockSpec((1, _D), lambda i: (0, 0)),
                  pl.BlockSpec((_OUT, _D), lambda i: (0, 0)),
                  pl.BlockSpec((1, _OUT), lambda i: (0, 0))],
        out_specs=(pl.BlockSpec((_BR, _D), lambda i: (i, 0)),
                   pl.BlockSpec((_BR, _OUT), lambda i: (i, 0))),
        out_shape=(jax.ShapeDtypeStruct((_N, _D), jnp.float32),
                   jax.ShapeDtypeStruct((_N, _OUT), jnp.float32)),
    )(h, ss4, cnt, Wl, bl.reshape(1, _D), Wr, scale.reshape(1, _D),
      shift.reshape(1, _D), Wh, bh.reshape(1, _OUT))


def _head_body(h_ref, w_ref, b_ref, o_ref):
    y = lax.dot_general(h_ref[...], w_ref[...], (((1,), (1,)), ((), ())),
                        preferred_element_type=jnp.float32)
    o_ref[...] = y + b_ref[...]


def _head(h, Wh, bh):
    return pl.pallas_call(
        _head_body,
        grid=(_GRID,),
        in_specs=[pl.BlockSpec((_BR, _D), lambda i: (i, 0)),
                  pl.BlockSpec((_OUT, _D), lambda i: (0, 0)),
                  pl.BlockSpec((1, _OUT), lambda i: (0, 0))],
        out_specs=pl.BlockSpec((_BR, _OUT), lambda i: (i, 0)),
        out_shape=jax.ShapeDtypeStruct((_N, _OUT), jnp.float32),
    )(h, Wh, bh.reshape(1, _OUT))


def _prep_edges(src, dst):
    pad = _EPAD - _E
    srcp = jnp.concatenate([src, jnp.zeros((pad,), jnp.int32)])
    dstp = jnp.concatenate([dst, jnp.full((pad,), _N, jnp.int32)])
    s4 = (srcp[None, :] * 4
          + jnp.arange(4, dtype=jnp.int32)[:, None]).reshape(4, _CHUNKS, _CH)
    return s4, dstp.reshape(_CHUNKS, _CH)


def kernel(x_user, x_item, edge_index_ui, edge_index_iu, Wp_user, bp_user,
           Wp_item, bp_item, Wl0, bl0, Wr0, gamma0, beta0, Wl1, bl1, Wr1,
           gamma1, beta1, Wh, bh):
    ei_ui = edge_index_ui.astype(jnp.int32)
    ei_iu = edge_index_iu.astype(jnp.int32)
    s4_iu, d_iu = _prep_edges(ei_iu[0], ei_iu[1])
    s4_ui, d_ui = _prep_edges(ei_ui[0], ei_ui[1])
    dcat = jnp.stack([d_iu, d_ui])

    ones16 = jnp.ones((_CH, 16), jnp.float32)
    z16 = jnp.zeros((_RPT, 16), jnp.float32)
    zb = jnp.zeros((_RPT, 32), jnp.bfloat16)

    cnt = _sc_count(dcat, ones16, z16)
    cnt_u, cnt_i = cnt[0], cnt[1]

    hu, hub = _proj(x_user, Wp_user, bp_user)
    hi, hib = _proj(x_item, Wp_item, bp_item)

    inv = 1.0 / jnp.sqrt(1.0 + _EPS)
    one = jnp.ones((_D,), jnp.float32)
    zero = jnp.zeros((_D,), jnp.float32)

    ssum_u = _sc_seg(hib.reshape(_N * 4, 32), s4_iu, d_iu, zb)
    ssum_i = _sc_seg(hub.reshape(_N * 4, 32), s4_ui, d_ui, zb)
    hu, hub = _layer(hu, ssum_u, cnt_u, Wl0, bl0, Wr0, gamma0 * inv, beta0)
    hi, hib = _layer(hi, ssum_i, cnt_i, Wl0, bl0, Wr0, one, zero)

    ssum_u = _sc_seg(hib.reshape(_N * 4, 32), s4_iu, d_iu, zb)
    hu, out = _layer_head(hu, ssum_u, cnt_u, Wl1, bl1, Wr1, gamma1 * inv,
                          beta1, Wh, bh)
    return out, hu


# speedup vs baseline: 1.4428x; 1.0021x over previous
import functools

import jax
import jax.numpy as jnp
from jax import lax
from jax.experimental import pallas as pl
from jax.experimental.pallas import tpu as pltpu
from jax.experimental.pallas import tpu_sc as plsc

_N = 50000
_E = 300000
_D = 128
_OUT = 64
_EPS = 1e-5

_NC, _NS = 2, 16
_CH = 128
_CHUNKS = -(-_E // (_NS * _CH)) * _NS
_EPAD = _CHUNKS * _CH
_CPT = _CHUNKS // _NS
_K = 7
_NG = _CPT // _K
_NPAD = 50048
_RPT = _NPAD // _NS

_BR = 512
_GRID = -(-_N // _BR)

_mesh = plsc.VectorSubcoreMesh(
    core_axis_name="c", subcore_axis_name="s", num_cores=_NC, num_subcores=_NS
)
_sc_params = pltpu.CompilerParams(use_tc_tiling_on_sc=False)


@functools.partial(
    pl.kernel,
    out_type=jax.ShapeDtypeStruct((2, _NPAD, _D), jnp.float32),
    mesh=_mesh,
    scratch_types=[
        pltpu.VMEM_SHARED((_NPAD, 16), jnp.float32),
        pltpu.VMEM((_CH, 16), jnp.float32),
        pltpu.VMEM((_CPT, _CH), jnp.int32),
        pltpu.SemaphoreType.DMA,
    ],
    compiler_params=_sc_params,
)
def _sc_count(dcat, ones_h, z16, cnt_out, acc, ones_v, didx, sem):
    c = lax.axis_index("c")
    s = lax.axis_index("s")
    pltpu.sync_copy(ones_h, ones_v)
    pltpu.sync_copy(dcat.at[c, pl.ds(s * _CPT, _CPT)], didx)
    pltpu.sync_copy(z16, acc.at[pl.ds(s * _RPT, _RPT)])
    plsc.subcore_barrier()

    def body(t, carry):
        for b in range(_K):
            pltpu.async_copy(ones_v, acc.at[didx.at[t * _K + b]], sem,
                             add=True)
        for b in range(_K):
            pltpu.make_async_copy(z16.at[pl.ds(0, _CH)], ones_v, sem).wait()
        return carry

    lax.fori_loop(0, _NG, body, 0)
    plsc.subcore_barrier()
    pltpu.sync_copy(acc.at[pl.ds(s * _RPT, _RPT)],
                    cnt_out.at[c, pl.ds(s * _RPT, _RPT), pl.ds(0, 16)])


@functools.partial(
    pl.kernel,
    out_type=jax.ShapeDtypeStruct((_NPAD, _D), jnp.bfloat16),
    mesh=_mesh,
    scratch_types=[
        pltpu.VMEM_SHARED((_NPAD, 32), jnp.bfloat16),
        pltpu.VMEM((_RPT // 4, 32), jnp.bfloat16),
        pltpu.VMEM((_CPT, _CH), jnp.int32),
        pltpu.VMEM((_CPT, _CH), jnp.int32),
        [pltpu.VMEM((_CH, 32), jnp.bfloat16)] * (2 * _K),
        [pltpu.SemaphoreType.DMA] * 4,
    ],
    compiler_params=_sc_params,
)
def _sc_seg(table, s4, dd, zb, out,
            acc, zbuf, sidx, didx, rows, sems):
    c = lax.axis_index("c")
    s = lax.axis_index("s")
    rows_ab = (rows[:_K], rows[_K:])
    gsem = (sems[0], sems[1])
    ssem = (sems[2], sems[3])
    pltpu.sync_copy(zb.at[pl.ds(0, _RPT // 4)], zbuf)

    def fire_g(table, g, ab):
        for b in range(_K):
            pltpu.async_copy(table.at[sidx.at[g * _K + b]], rows_ab[ab][b],
                             gsem[ab])

    def drain_g(ab):
        for b in range(_K):
            pltpu.make_async_copy(zb.at[pl.ds(0, _CH)], rows_ab[ab][b],
                                  gsem[ab]).wait()

    def fire_s(g, ab):
        for b in range(_K):
            pltpu.async_copy(rows_ab[ab][b], acc.at[didx.at[g * _K + b]],
                             ssem[ab], add=True)

    def drain_s(ab):
        for b in range(_K):
            pltpu.make_async_copy(zb.at[pl.ds(0, _CH)], rows_ab[ab][b],
                                  ssem[ab]).wait()

    pltpu.sync_copy(dd.at[pl.ds(s * _CPT, _CPT)], didx)
    for p_loc in range(2):
        p = c * 2 + p_loc
        pltpu.sync_copy(s4.at[p, pl.ds(s * _CPT, _CPT)], sidx)
        fire_g(table, 0, 0)
        fire_g(table, 1, 1)
        for z in range(4):
            pltpu.sync_copy(
                zbuf,
                acc.at[pl.ds(s * _RPT + z * (_RPT // 4), _RPT // 4)])
        plsc.subcore_barrier()

        drain_g(0)
        fire_s(0, 0)

        def pair(t, carry):
            for par in range(2):
                g = 2 * t + 1 + par
                ab = (1, 0)[par]
                drain_g(ab)
                fire_s(g, ab)
                drain_s(1 - ab)
                fire_g(table, g + 1, 1 - ab)
            return carry

        lax.fori_loop(0, (_NG - 3) // 2, pair, 0)
        drain_g(1)
        fire_s(_NG - 2, 1)
        drain_s(0)
        fire_g(table, _NG - 1, 0)
        drain_g(0)
        fire_s(_NG - 1, 0)
        drain_s(1)
        drain_s(0)
        plsc.subcore_barrier()
        pltpu.sync_copy(acc.at[pl.ds(s * _RPT, _RPT)],
                        out.at[pl.ds(s * _RPT, _RPT), pl.ds(p * 32, 32)])


def _proj_body(x_ref, w_ref, b_ref, o_ref, ob_ref):
    y = lax.dot_general(x_ref[...], w_ref[...], (((1,), (1,)), ((), ())),
                        preferred_element_type=jnp.float32)
    y = jnp.maximum(y + b_ref[...], 0.0)
    o_ref[...] = y
    ob_ref[...] = y.astype(jnp.bfloat16)


def _proj(x, W, b):
    return pl.pallas_call(
        _proj_body,
        grid=(_GRID,),
        in_specs=[pl.BlockSpec((_BR, _D), lambda i: (i, 0)),
                  pl.BlockSpec((_D, _D), lambda i: (0, 0)),
                  pl.BlockSpec((1, _D), lambda i: (0, 0))],
        out_specs=(pl.BlockSpec((_BR, _D), lambda i: (i, 0)),
                   pl.BlockSpec((_BR, _D), lambda i: (i, 0))),
        out_shape=(jax.ShapeDtypeStruct((_N, _D), jnp.float32),
                   jax.ShapeDtypeStruct((_N, _D), jnp.bfloat16)),
    )(x, W, b.reshape(1, _D))


def _layer_body(h_ref, ss_ref, cnt_ref, wl_ref, bl_ref, wr_ref, sc_ref,
                sh_ref, o_ref, ob_ref=None):
    h = h_ref[...]
    mean = (ss_ref[...].astype(jnp.float32)
            / jnp.maximum(cnt_ref[...][:, :1], 1.0))
    agg = lax.dot_general(mean, wl_ref[...], (((1,), (1,)), ((), ())),
                          preferred_element_type=jnp.float32)
    agg += lax.dot_general(h, wr_ref[...], (((1,), (1,)), ((), ())),
                           preferred_element_type=jnp.float32)
    t = (h + agg + bl_ref[...]) * 0.5
    y = jnp.maximum(t * sc_ref[...] + sh_ref[...], 0.0)
    o_ref[...] = y
    if ob_ref is not None:
        ob_ref[...] = y.astype(jnp.bfloat16)


def _layer(h, ss4, cnt, Wl, bl, Wr, scale, shift, want_bf16=True):
    if want_bf16:
        out_specs = (pl.BlockSpec((_BR, _D), lambda i: (i, 0)),
                     pl.BlockSpec((_BR, _D), lambda i: (i, 0)))
        out_shape = (jax.ShapeDtypeStruct((_N, _D), jnp.float32),
                     jax.ShapeDtypeStruct((_N, _D), jnp.bfloat16))
    else:
        out_specs = pl.BlockSpec((_BR, _D), lambda i: (i, 0))
        out_shape = jax.ShapeDtypeStruct((_N, _D), jnp.float32)
    return pl.pallas_call(
        _layer_body,
        grid=(_GRID,),
        in_specs=[pl.BlockSpec((_BR, _D), lambda i: (i, 0)),
                  pl.BlockSpec((_BR, _D), lambda i: (i, 0)),
                  pl.BlockSpec((_BR, _D), lambda i: (i, 0)),
                  pl.BlockSpec((_D, _D), lambda i: (0, 0)),
                  pl.BlockSpec((1, _D), lambda i: (0, 0)),
                  pl.BlockSpec((_D, _D), lambda i: (0, 0)),
                  pl.BlockSpec((1, _D), lambda i: (0, 0)),
                  pl.BlockSpec((1, _D), lambda i: (0, 0))],
        out_specs=out_specs,
        out_shape=out_shape,
    )(h, ss4, cnt, Wl, bl.reshape(1, _D), Wr, scale.reshape(1, _D),
      shift.reshape(1, _D))


def _layer_head_body(h_ref, ss_ref, cnt_ref, wl_ref, bl_ref, wr_ref,
                     sc_ref, sh_ref, wh_ref, bh_ref, o_ref, oo_ref):
    h = h_ref[...]
    mean = (ss_ref[...].astype(jnp.float32)
            / jnp.maximum(cnt_ref[...][:, :1], 1.0))
    agg = lax.dot_general(mean, wl_ref[...], (((1,), (1,)), ((), ())),
                          preferred_element_type=jnp.float32)
    agg += lax.dot_general(h, wr_ref[...], (((1,), (1,)), ((), ())),
                           preferred_element_type=jnp.float32)
    t = (h + agg + bl_ref[...]) * 0.5
    y = jnp.maximum(t * sc_ref[...] + sh_ref[...], 0.0)
    o_ref[...] = y
    oo_ref[...] = lax.dot_general(
        y, wh_ref[...], (((1,), (1,)), ((), ())),
        preferred_element_type=jnp.float32) + bh_ref[...]


def _layer_head(h, ss4, cnt, Wl, bl, Wr, scale, shift, Wh, bh):
    return pl.pallas_call(
        _layer_head_body,
        grid=(_GRID,),
        in_specs=[pl.BlockSpec((_BR, _D), lambda i: (i, 0)),
                  pl.BlockSpec((_BR, _D), lambda i: (i, 0)),
                  pl.BlockSpec((_BR, _D), lambda i: (i, 0)),
                  pl.BlockSpec((_D, _D), lambda i: (0, 0)),
                  pl.BlockSpec((1, _D), lambda i: (0, 0)),
                  pl.BlockSpec((_D, _D), lambda i: (0, 0)),
                  pl.BlockSpec((1, _D), lambda i: (0, 0)),
                  pl.BlockSpec((1, _D), lambda i: (0, 0)),
                  pl.BlockSpec((_OUT, _D), lambda i: (0, 0)),
                  pl.BlockSpec((1, _OUT), lambda i: (0, 0))],
        out_specs=(pl.BlockSpec((_BR, _D), lambda i: (i, 0)),
                   pl.BlockSpec((_BR, _OUT), lambda i: (i, 0))),
        out_shape=(jax.ShapeDtypeStruct((_N, _D), jnp.float32),
                   jax.ShapeDtypeStruct((_N, _OUT), jnp.float32)),
    )(h, ss4, cnt, Wl, bl.reshape(1, _D), Wr, scale.reshape(1, _D),
      shift.reshape(1, _D), Wh, bh.reshape(1, _OUT))


def _head_body(h_ref, w_ref, b_ref, o_ref):
    y = lax.dot_general(h_ref[...], w_ref[...], (((1,), (1,)), ((), ())),
                        preferred_element_type=jnp.float32)
    o_ref[...] = y + b_ref[...]


def _head(h, Wh, bh):
    return pl.pallas_call(
        _head_body,
        grid=(_GRID,),
        in_specs=[pl.BlockSpec((_BR, _D), lambda i: (i, 0)),
                  pl.BlockSpec((_OUT, _D), lambda i: (0, 0)),
                  pl.BlockSpec((1, _OUT), lambda i: (0, 0))],
        out_specs=pl.BlockSpec((_BR, _OUT), lambda i: (i, 0)),
        out_shape=jax.ShapeDtypeStruct((_N, _OUT), jnp.float32),
    )(h, Wh, bh.reshape(1, _OUT))


def _prep_edges(src, dst):
    pad = _EPAD - _E
    srcp = jnp.concatenate([src, jnp.zeros((pad,), jnp.int32)])
    dstp = jnp.concatenate([dst, jnp.full((pad,), _N, jnp.int32)])
    s4 = (srcp[None, :] * 4
          + jnp.arange(4, dtype=jnp.int32)[:, None]).reshape(4, _CHUNKS, _CH)
    return s4, dstp.reshape(_CHUNKS, _CH)


def kernel(x_user, x_item, edge_index_ui, edge_index_iu, Wp_user, bp_user,
           Wp_item, bp_item, Wl0, bl0, Wr0, gamma0, beta0, Wl1, bl1, Wr1,
           gamma1, beta1, Wh, bh):
    ei_ui = edge_index_ui.astype(jnp.int32)
    ei_iu = edge_index_iu.astype(jnp.int32)
    s4_iu, d_iu = _prep_edges(ei_iu[0], ei_iu[1])
    s4_ui, d_ui = _prep_edges(ei_ui[0], ei_ui[1])
    dcat = jnp.stack([d_iu, d_ui])

    ones16 = jnp.ones((_CH, 16), jnp.float32)
    z16 = jnp.zeros((_RPT, 16), jnp.float32)
    zb = jnp.zeros((_RPT, 32), jnp.bfloat16)

    cnt = _sc_count(dcat, ones16, z16)
    cnt_u, cnt_i = cnt[0], cnt[1]

    hu, hub = _proj(x_user, Wp_user, bp_user)
    hi, hib = _proj(x_item, Wp_item, bp_item)

    inv = 1.0 / jnp.sqrt(1.0 + _EPS)
    one = jnp.ones((_D,), jnp.float32)
    zero = jnp.zeros((_D,), jnp.float32)

    ssum_u = _sc_seg(hib.reshape(_N * 4, 32), s4_iu, d_iu, zb)
    ssum_i = _sc_seg(hub.reshape(_N * 4, 32), s4_ui, d_ui, zb)
    hu = _layer(hu, ssum_u, cnt_u, Wl0, bl0, Wr0, gamma0 * inv, beta0,
                want_bf16=False)
    hi, hib = _layer(hi, ssum_i, cnt_i, Wl0, bl0, Wr0, one, zero)

    ssum_u = _sc_seg(hib.reshape(_N * 4, 32), s4_iu, d_iu, zb)
    hu, out = _layer_head(hu, ssum_u, cnt_u, Wl1, bl1, Wr1, gamma1 * inv,
                          beta1, Wh, bh)
    return out, hu
